# Initial kernel scaffold; baseline (speedup 1.0000x reference)
import jax
import jax.numpy as jnp
from jax.experimental import pallas as pl


def kernel(x, params):
    raise NotImplementedError("write your pallas kernel here")


# speedup vs baseline: 11.6713x; 11.6713x over previous
import functools
import math

import jax
import jax.numpy as jnp
from jax import lax
from jax.experimental import pallas as pl
from jax.experimental.pallas import tpu as pltpu
from jax.experimental.pallas import tpu_sc as plsc

_BN = 1.0 / math.sqrt(1.0 + 1e-5)


def _rup(n, m):
    return -(-n // m) * m


def _fps_body(planes_ref, rows_ref, out_ref, *, npoint, n_real):
    nr, nc = planes_ref.shape[1], planes_ref.shape[2]
    xpl = planes_ref[0]
    ypl = planes_ref[1]
    zpl = planes_ref[2]
    flat = (lax.broadcasted_iota(jnp.int32, (nr, nc), 0) * nc
            + lax.broadcasted_iota(jnp.int32, (nr, nc), 1))
    dist0 = jnp.where(flat < n_real, jnp.float32(1e10), jnp.float32(-1.0))

    def body(i, carry):
        dist, far = carry
        row = rows_ref[pl.ds(far, 1), :]
        out_ref[pl.ds(i, 1), :] = row
        cx = row[0:1, 0:1]
        cy = row[0:1, 1:2]
        cz = row[0:1, 2:3]
        dx = xpl - cx
        dy = ypl - cy
        dz = zpl - cz
        d = dx * dx + dy * dy + dz * dz
        dist = jnp.minimum(dist, d)
        m = jnp.max(dist)
        far2 = jnp.min(jnp.where(dist == m, flat, jnp.int32(2 ** 30)))
        return dist, far2

    lax.fori_loop(0, npoint, body, (dist0, jnp.int32(0)))


def _fps(planes, rows128, npoint, n_real, sp):
    nr, nc = planes.shape[1], planes.shape[2]
    nrows = rows128.shape[0]
    return pl.pallas_call(
        functools.partial(_fps_body, npoint=npoint, n_real=n_real),
        out_shape=jax.ShapeDtypeStruct((sp, 128), jnp.float32),
        in_specs=[pl.BlockSpec((3, nr, nc), lambda: (0, 0, 0)),
                  pl.BlockSpec((nrows, 128), lambda: (0, 0))],
        out_specs=pl.BlockSpec((sp, 128), lambda: (0, 0)),
    )(planes, rows128)


def _sqdists(q, t_ref):
    qx = q[:, 0:1]
    qy = q[:, 1:2]
    qz = q[:, 2:3]
    px = t_ref[0:1, :]
    py = t_ref[1:2, :]
    pz = t_ref[2:3, :]
    s1 = qx * qx + qy * qy + qz * qz
    s2 = px * px + py * py + pz * pz
    g = qx * px + qy * py + qz * pz
    return s1 + s2 - 2.0 * g


def _ballq_body(q_ref, t_ref, o_ref, *, k, n_real, r2):
    sqr = _sqdists(q_ref[...], t_ref)
    iotaf = lax.broadcasted_iota(jnp.int32, sqr.shape, 1).astype(jnp.float32)
    keys = jnp.where(sqr > r2, jnp.float32(n_real), iotaf)
    cols = []
    for _ in range(k):
        m = jnp.min(keys, axis=1, keepdims=True)
        cols.append(m)
        keys = jnp.where(keys == m, jnp.float32(1e9), keys)
    idx = jnp.concatenate(cols, axis=1)
    first = idx[:, 0:1]
    idx = jnp.where(idx >= jnp.float32(n_real), first, idx)
    idx = jnp.minimum(idx, jnp.float32(n_real - 1))
    pad = jnp.zeros((idx.shape[0], 128 - k), jnp.float32)
    o_ref[...] = jnp.concatenate([idx, pad], axis=1).astype(jnp.int32)


def _ballq(q, tbl, r2, k, n_real):
    sp = q.shape[0]
    npc = tbl.shape[1]
    bs = 128
    return pl.pallas_call(
        functools.partial(_ballq_body, k=k, n_real=n_real, r2=r2),
        grid=(sp // bs,),
        out_shape=jax.ShapeDtypeStruct((sp, 128), jnp.int32),
        in_specs=[pl.BlockSpec((bs, 3), lambda i: (i, 0)),
                  pl.BlockSpec((3, npc), lambda i: (0, 0))],
        out_specs=pl.BlockSpec((bs, 128), lambda i: (i, 0)),
    )(q, tbl)


def _smallk_mm(x, w):
    y = x[:, 0:1] * w[0:1, :]
    for j in range(1, x.shape[1]):
        y = y + x[:, j:j + 1] * w[j:j + 1, :]
    return y


def _dense_body(x_ref, w_ref, b_ref, o_ref, *, mode):
    x = x_ref[...]
    w = w_ref[...]
    if x.shape[1] <= 8:
        y = _smallk_mm(x, w)
    else:
        y = jnp.dot(x, w, preferred_element_type=jnp.float32)
    if mode == "relu_bn":
        y = jnp.maximum((y + b_ref[...]) * _BN, 0.0)
    elif mode == "head1":
        y = y + b_ref[...]
        y = jnp.where(y > 0, y, 0.2 * y) * _BN
    o_ref[...] = y


def _dense(x, wt, b, mode):
    m, kdim = x.shape
    c = wt.shape[1]
    if b is None:
        b = jnp.zeros((c,), jnp.float32)
    bm = 128
    return pl.pallas_call(
        functools.partial(_dense_body, mode=mode),
        grid=(m // bm,),
        out_shape=jax.ShapeDtypeStruct((m, c), jnp.float32),
        in_specs=[pl.BlockSpec((bm, kdim), lambda i: (i, 0)),
                  pl.BlockSpec((kdim, c), lambda i: (0, 0)),
                  pl.BlockSpec((1, c), lambda i: (0, 0))],
        out_specs=pl.BlockSpec((bm, c), lambda i: (i, 0)),
    )(x, wt, b.reshape(1, c))


def _sc_gather(table, idx, d):
    bp = idx.shape[0]
    ch = 128
    bw = bp // 32
    nl = bw // ch
    mesh = plsc.VectorSubcoreMesh(core_axis_name="c", subcore_axis_name="s")

    @functools.partial(
        pl.kernel, mesh=mesh,
        out_type=jax.ShapeDtypeStruct((bp, d), jnp.float32),
        scratch_types=[pltpu.VMEM((ch,), jnp.int32),
                       pltpu.VMEM((ch, d), jnp.float32),
                       pltpu.SemaphoreType.DMA],
    )
    def body(t_hbm, i_hbm, o_hbm, idx_v, rows_v, sem):
        wid = lax.axis_index("s") * 2 + lax.axis_index("c")
        base0 = wid * bw
        for j in range(nl):
            base = base0 + j * ch
            pltpu.sync_copy(i_hbm.at[pl.ds(base, ch)], idx_v)
            pltpu.async_copy(t_hbm.at[idx_v], rows_v, sem).wait()
            pltpu.sync_copy(rows_v, o_hbm.at[pl.ds(base, ch)])

    return body(table, idx)


def _gather_sliced(a, idxm, k, sp):
    idxf = idxm[:, :k].reshape(-1)
    bp = _rup(sp * k, 4096)
    idxf = jnp.pad(idxf, (0, bp - sp * k))
    return _sc_gather(a, idxf, a.shape[1])[: sp * k]


def _query_offset(pd, wd, b):
    if pd.shape[1] <= 8:
        d = _smallk_mm(pd, wd)
    else:
        d = jnp.dot(pd, wd, preferred_element_type=jnp.float32)
    return d + b


def _k4a_body(g_ref, pd_ref, wd_ref, b_ref, o_ref, *, k):
    qt = pd_ref.shape[0]
    c = g_ref.shape[1]
    gmax = jnp.max(g_ref[...].reshape(qt, k, c), axis=1)
    d = _query_offset(pd_ref[...], wd_ref[...], b_ref[...])
    o_ref[...] = jnp.maximum((gmax + d) * _BN, 0.0)


def _k4a(g, pd, wd, b, k):
    sp, kd = pd.shape
    c = g.shape[1]
    qt = 128
    return pl.pallas_call(
        functools.partial(_k4a_body, k=k),
        grid=(sp // qt,),
        out_shape=jax.ShapeDtypeStruct((sp, c), jnp.float32),
        in_specs=[pl.BlockSpec((qt * k, c), lambda i: (i, 0)),
                  pl.BlockSpec((qt, kd), lambda i: (i, 0)),
                  pl.BlockSpec((kd, c), lambda i: (0, 0)),
                  pl.BlockSpec((1, c), lambda i: (0, 0))],
        out_specs=pl.BlockSpec((qt, c), lambda i: (i, 0)),
    )(g, pd, wd, b.reshape(1, c))


def _k4b_body(g_ref, pd_ref, wd_ref, b_ref, w2_ref, b2_ref, o_ref, *, k):
    qt = pd_ref.shape[0]
    c1 = g_ref.shape[1]
    c2 = w2_ref.shape[1]
    d = _query_offset(pd_ref[...], wd_ref[...], b_ref[...])
    h = jnp.maximum((g_ref[...].reshape(qt, k, c1) + d[:, None, :]) * _BN, 0.0)
    y = jnp.dot(h.reshape(qt * k, c1), w2_ref[...],
                preferred_element_type=jnp.float32) + b2_ref[...]
    y = jnp.maximum(y * _BN, 0.0)
    o_ref[...] = jnp.max(y.reshape(qt, k, c2), axis=1)


def _k4b(g, pd, wd, b, w2t, b2, k):
    sp, kd = pd.shape
    c1 = g.shape[1]
    c2 = w2t.shape[1]
    qt = 128
    return pl.pallas_call(
        functools.partial(_k4b_body, k=k),
        grid=(sp // qt,),
        out_shape=jax.ShapeDtypeStruct((sp, c2), jnp.float32),
        in_specs=[pl.BlockSpec((qt * k, c1), lambda i: (i, 0)),
                  pl.BlockSpec((qt, kd), lambda i: (i, 0)),
                  pl.BlockSpec((kd, c1), lambda i: (0, 0)),
                  pl.BlockSpec((1, c1), lambda i: (0, 0)),
                  pl.BlockSpec((c1, c2), lambda i: (0, 0)),
                  pl.BlockSpec((1, c2), lambda i: (0, 0))],
        out_specs=pl.BlockSpec((qt, c2), lambda i: (i, 0)),
    )(g, pd, wd, b.reshape(1, c1), w2t, b2.reshape(1, c2))


def _top3_body(q_ref, t_ref, oi_ref, ow_ref):
    d = _sqdists(q_ref[...], t_ref)
    iotaf = lax.broadcasted_iota(jnp.int32, d.shape, 1).astype(jnp.float32)
    idxs, ds = [], []
    for _ in range(3):
        m = jnp.min(d, axis=1, keepdims=True)
        ir = jnp.min(jnp.where(d == m, iotaf, jnp.float32(1e9)),
                     axis=1, keepdims=True)
        ds.append(m)
        idxs.append(ir)
        d = jnp.where(iotaf == ir, jnp.float32(1e30), d)
    w = [1.0 / (dd + 1e-8) for dd in ds]
    norm = (w[0] + w[1]) + w[2]
    w = [ww / norm for ww in w]
    pad = jnp.zeros((q_ref.shape[0], 125), jnp.float32)
    oi_ref[...] = jnp.concatenate(idxs + [pad], axis=1).astype(jnp.int32)
    ow_ref[...] = jnp.concatenate(w + [pad], axis=1)


def _top3(q, tbl):
    sp = q.shape[0]
    npc = tbl.shape[1]
    bs = 128
    return pl.pallas_call(
        _top3_body,
        grid=(sp // bs,),
        out_shape=(jax.ShapeDtypeStruct((sp, 128), jnp.int32),
                   jax.ShapeDtypeStruct((sp, 128), jnp.float32)),
        in_specs=[pl.BlockSpec((bs, 3), lambda i: (i, 0)),
                  pl.BlockSpec((3, npc), lambda i: (0, 0))],
        out_specs=(pl.BlockSpec((bs, 128), lambda i: (i, 0)),
                   pl.BlockSpec((bs, 128), lambda i: (i, 0))),
    )(q, tbl)


def _fp_comb_body(g_ref, w_ref, b_ref, o_ref):
    qt = w_ref.shape[0]
    c = g_ref.shape[1]
    g = g_ref[...].reshape(qt, 3, c)
    wv = w_ref[...]
    acc = (g[:, 0, :] * wv[:, 0:1] + g[:, 1, :] * wv[:, 1:2]) \
        + g[:, 2, :] * wv[:, 2:3]
    o_ref[...] = jnp.maximum((acc + b_ref[...]) * _BN, 0.0)


def _fp_comb(g, wm, b):
    sp = wm.shape[0]
    c = g.shape[1]
    qt = 128
    return pl.pallas_call(
        _fp_comb_body,
        grid=(sp // qt,),
        out_shape=jax.ShapeDtypeStruct((sp, c), jnp.float32),
        in_specs=[pl.BlockSpec((qt * 3, c), lambda i: (i, 0)),
                  pl.BlockSpec((qt, 128), lambda i: (i, 0)),
                  pl.BlockSpec((1, c), lambda i: (0, 0))],
        out_specs=pl.BlockSpec((qt, c), lambda i: (i, 0)),
    )(g, wm, b.reshape(1, c))


def _final_body(x_ref, w_ref, b_ref, r_ref, o_ref):
    y = jnp.dot(x_ref[...], w_ref[...], preferred_element_type=jnp.float32)
    o_ref[...] = y + b_ref[...] + r_ref[...]


def _final(x, wt, b, res):
    m = x.shape[0]
    c = wt.shape[1]
    bm = 128
    return pl.pallas_call(
        _final_body,
        grid=(m // bm,),
        out_shape=jax.ShapeDtypeStruct((m, c), jnp.float32),
        in_specs=[pl.BlockSpec((bm, x.shape[1]), lambda i: (i, 0)),
                  pl.BlockSpec((x.shape[1], c), lambda i: (0, 0)),
                  pl.BlockSpec((1, c), lambda i: (0, 0)),
                  pl.BlockSpec((bm, c), lambda i: (i, 0))],
        out_specs=pl.BlockSpec((bm, c), lambda i: (i, 0)),
    )(x, wt, b.reshape(1, c), res)


def _masked_table(p, n_real):
    colid = jnp.arange(p.shape[0])[None, :]
    return jnp.where(colid < n_real, jnp.transpose(p), jnp.float32(1e15))


def kernel(x, params):
    x0 = x[0]
    frames = [jnp.transpose(x0[t]) for t in range(4)]

    def sa(cloud, feats, n_real, npoint, radius, k, layers):
        nrows = cloud.shape[0]
        planes = jnp.transpose(cloud).reshape(3, 8, nrows // 8)
        rows128 = jnp.pad(cloud, ((0, 0), (0, 125)))
        sp = _rup(npoint, 128)
        raw = _fps(planes, rows128, npoint, n_real, sp)
        new_xyz = jnp.concatenate(
            [raw[:npoint, :3], jnp.zeros((sp - npoint, 3), jnp.float32)], axis=0)
        idxm = _ballq(new_xyz, _masked_table(cloud, n_real),
                      radius * radius, k, n_real)
        w1, b1 = layers[0]
        if feats is None:
            a = _dense(cloud, jnp.transpose(w1[:, :3]), None, "proj")
        else:
            a = _dense(jnp.concatenate([cloud, feats], axis=1),
                       jnp.transpose(w1), None, "proj")
        g = _gather_sliced(a, idxm, k, sp)
        wd = -jnp.transpose(w1[:, :3])
        if len(layers) == 2:
            w2, b2 = layers[1]
            ft = _k4b(g, new_xyz, wd, b1, jnp.transpose(w2), b2, k)
        else:
            ft = _k4a(g, new_xyz, wd, b1, k)
        return new_xyz, ft

    def fe(p1, f1, p2, f2, n2, radius, k, layer):
        w, b = layer
        c2 = f2.shape[1]
        wp, wf2, wf1 = w[:, :3], w[:, 3:3 + c2], w[:, 3 + c2:]
        a = _dense(jnp.concatenate([p2, f2], axis=1),
                   jnp.concatenate([jnp.transpose(wp), jnp.transpose(wf2)],
                                   axis=0), None, "proj")
        idxm = _ballq(p1, _masked_table(p2, n2), radius * radius, k, n2)
        g = _gather_sliced(a, idxm, k, p1.shape[0])
        pd = jnp.concatenate([p1, f1], axis=1)
        wd = jnp.concatenate([-jnp.transpose(wp), jnp.transpose(wf1)], axis=0)
        return _k4a(g, pd, wd, b, k)

    def su(p1, f1, p2, f2, n2, radius, k, layer, layer2):
        w, b = layer
        c2 = f2.shape[1]
        wf, wp = w[:, :c2], w[:, c2:]
        a = _dense(jnp.concatenate([f2, p2], axis=1),
                   jnp.concatenate([jnp.transpose(wf), jnp.transpose(wp)],
                                   axis=0), None, "proj")
        idxm = _ballq(p1, _masked_table(p2, n2), radius * radius, k, n2)
        g = _gather_sliced(a, idxm, k, p1.shape[0])
        ftn = _k4a(g, p1, -jnp.transpose(wp), b, k)
        w2, b2 = layer2
        return _dense(jnp.concatenate([ftn, f1], axis=1),
                      jnp.transpose(w2), b2, "relu_bn")

    def fp(p1, p2, f2, n2, layer):
        w, b = layer
        a = _dense(f2, jnp.transpose(w[:, :f2.shape[1]]), None, "proj")
        idxm, wm = _top3(p1, _masked_table(p2, n2))
        idxf = idxm[:, :3].reshape(-1)
        g = _sc_gather(a, idxf, a.shape[1])
        return _fp_comb(g, wm, b)

    pc0, ft0 = sa(frames[0], None, 8192, 5500, 0.5, 16, params["sa1"])
    pc1, ft1 = sa(frames[1], None, 8192, 5500, 0.5, 16, params["sa1"])
    pc2, ft2 = sa(frames[2], None, 8192, 5500, 0.5, 16, params["sa1"])
    pc3, ft3 = sa(frames[3], None, 8192, 5500, 0.5, 16, params["sa1"])

    fe1a = fe(pc1, ft1, pc0, ft0, 5500, 1.5, 24, params["fe1"][0])
    fe1b = fe(pc3, ft3, pc2, ft2, 5500, 1.5, 24, params["fe1"][0])

    qc0, qf0 = sa(pc1, fe1a, 5500, 1375, 1.0, 16, params["sa2"])
    qc1, qf1 = sa(pc3, fe1b, 5500, 1375, 1.0, 16, params["sa2"])

    l2b = fe(qc1, qf1, qc0, qf0, 1375, 3.0, 24, params["fe2"][0])
    rc, rf = sa(qc1, l2b, 1375, 275, 2.0, 16, params["sa3"])

    l2f = su(qc1, qf1, rc, rf, 275, 4.0, 16,
             params["su1_mlp"][0], params["su1_mlp2"][0])
    l1f = su(pc3, ft3, qc1, l2f, 1375, 2.0, 16,
             params["su2_mlp"][0], params["su2_mlp2"][0])

    p0 = frames[3]
    l0 = fp(p0, pc3, l1f, 5500, params["fp"][0])

    (hw1, hb1), (hw2, hb2) = params["head"]
    y = _dense(l0, jnp.transpose(hw1), hb1, "head1")
    w2p = jnp.pad(jnp.transpose(hw2), ((0, 0), (0, 5)))
    b2p = jnp.pad(hb2, (0, 5))
    resp = jnp.pad(p0, ((0, 0), (0, 5)))
    outr = _final(y, w2p, b2p, resp)
    return jnp.transpose(outr[:, :3])[None]
